# baseline (speedup 1.0000x reference)
import jax
import jax.numpy as jnp
from jax.experimental import pallas as pl
from jax.experimental.pallas import tpu as pltpu

N = 10000
DIN = 128
FC = 256
G = 64

NCH = 10
CH = N // NCH
NBUF = 4


def _pool_kernel(x_hbm, batch_ref, w_ref, b_ref, out_ref, bufs, sems):
    ids = jax.lax.broadcasted_iota(jnp.int32, (1, G), 1)

    def start(c):
        pltpu.make_async_copy(x_hbm.at[pl.ds(c * CH, CH)],
                              bufs.at[c % NBUF], sems.at[c % NBUF]).start()

    for c in range(NBUF):
        start(c)

    sums = jnp.zeros((G, DIN), jnp.float32)
    cnt = jnp.zeros((1, G), jnp.float32)
    for c in range(NCH):
        pltpu.make_async_copy(x_hbm.at[pl.ds(c * CH, CH)],
                              bufs.at[c % NBUF], sems.at[c % NBUF]).wait()
        seg = (batch_ref[pl.ds(c * CH, CH), :] == ids).astype(jnp.float32)
        sums += jax.lax.dot_general(seg, bufs[c % NBUF],
                                    (((0,), (0,)), ((), ())),
                                    preferred_element_type=jnp.float32)
        cnt += jnp.sum(seg, axis=0, keepdims=True)
        if c + NBUF < NCH:
            start(c + NBUF)

    pooled = sums / jnp.maximum(cnt, 1.0).T
    out_ref[...] = jnp.dot(pooled, w_ref[...],
                           preferred_element_type=jnp.float32) + b_ref[...]


def kernel(x, edge_index, edge_attr, batch, W_atom, b_atom, W_edge, b_edge):
    del edge_index, edge_attr, W_edge, b_edge
    return pl.pallas_call(
        _pool_kernel,
        in_specs=[
            pl.BlockSpec(memory_space=pltpu.HBM),
            pl.BlockSpec(memory_space=pltpu.VMEM),
            pl.BlockSpec(memory_space=pltpu.VMEM),
            pl.BlockSpec(memory_space=pltpu.VMEM),
        ],
        out_specs=pl.BlockSpec(memory_space=pltpu.VMEM),
        out_shape=jax.ShapeDtypeStruct((G, FC), jnp.float32),
        scratch_shapes=[
            pltpu.VMEM((NBUF, CH, DIN), jnp.float32),
            pltpu.SemaphoreType.DMA((NBUF,)),
        ],
    )(x, batch.reshape(N, 1), W_atom, b_atom.reshape(1, FC))


# speedup vs baseline: 1.2073x; 1.2073x over previous
import jax
import jax.numpy as jnp
from jax.experimental import pallas as pl
from jax.experimental.pallas import tpu as pltpu

N = 10000
DIN = 128
FC = 256
G = 64


def _pool_kernel(x_hbm, batch_ref, w_ref, b_ref, out_ref, buf, sem):
    pltpu.make_async_copy(x_hbm, buf, sem).start()
    ids = jax.lax.broadcasted_iota(jnp.int32, (N, G), 1).astype(jnp.int8)
    seg = (batch_ref[...] == ids).astype(jnp.float32)
    cnt = jnp.sum(seg, axis=0, keepdims=True)
    pltpu.make_async_copy(x_hbm, buf, sem).wait()
    sums = jax.lax.dot_general(seg, buf[...], (((0,), (0,)), ((), ())),
                               preferred_element_type=jnp.float32)
    pooled = sums / jnp.maximum(cnt, 1.0).T
    out_ref[...] = jnp.dot(pooled, w_ref[...],
                           preferred_element_type=jnp.float32) + b_ref[...]


def kernel(x, edge_index, edge_attr, batch, W_atom, b_atom, W_edge, b_edge):
    del edge_index, edge_attr, W_edge, b_edge
    batch_b = jnp.broadcast_to(batch.astype(jnp.int8)[:, None], (N, G))
    return pl.pallas_call(
        _pool_kernel,
        in_specs=[
            pl.BlockSpec(memory_space=pltpu.HBM),
            pl.BlockSpec(memory_space=pltpu.VMEM),
            pl.BlockSpec(memory_space=pltpu.VMEM),
            pl.BlockSpec(memory_space=pltpu.VMEM),
        ],
        out_specs=pl.BlockSpec(memory_space=pltpu.VMEM),
        out_shape=jax.ShapeDtypeStruct((G, FC), jnp.float32),
        scratch_shapes=[
            pltpu.VMEM((N, DIN), jnp.float32),
            pltpu.SemaphoreType.DMA,
        ],
    )(x, batch_b, W_atom, b_atom.reshape(1, FC))


# speedup vs baseline: 1.9129x; 1.5845x over previous
import jax
import jax.numpy as jnp
from jax.experimental import pallas as pl
from jax.experimental.pallas import tpu as pltpu

N = 10000
DIN = 128
FC = 256
G = 64


def _pool_kernel(x_hbm, batch_ref, w_ref, b_ref, out_ref, buf, sem):
    pltpu.make_async_copy(x_hbm, buf, sem).start()
    bat = batch_ref[...]
    lane = jax.lax.broadcasted_iota(jnp.int32, (1, G), 1)
    ends = jnp.zeros((1, G), jnp.int32)
    starts = jnp.zeros((1, G), jnp.int32)
    for g in range(G):
        sel = (lane == g).astype(jnp.int32)
        ends = ends + jnp.sum((bat <= g).astype(jnp.int32)) * sel
        starts = starts + jnp.sum((bat < g).astype(jnp.int32)) * sel
    riota = jax.lax.broadcasted_iota(jnp.int32, (N, G), 0)
    seg = ((riota >= starts) & (riota < ends)).astype(jnp.float32)
    cnt = (ends - starts).astype(jnp.float32)
    pltpu.make_async_copy(x_hbm, buf, sem).wait()
    sums = jax.lax.dot_general(seg, buf[...], (((0,), (0,)), ((), ())),
                               preferred_element_type=jnp.float32)
    pooled = sums / jnp.maximum(cnt, 1.0).T
    out_ref[...] = jnp.dot(pooled, w_ref[...],
                           preferred_element_type=jnp.float32) + b_ref[...]


def kernel(x, edge_index, edge_attr, batch, W_atom, b_atom, W_edge, b_edge):
    del edge_index, edge_attr, W_edge, b_edge
    batch2d = batch.reshape(80, 125)
    return pl.pallas_call(
        _pool_kernel,
        in_specs=[
            pl.BlockSpec(memory_space=pltpu.HBM),
            pl.BlockSpec(memory_space=pltpu.VMEM),
            pl.BlockSpec(memory_space=pltpu.VMEM),
            pl.BlockSpec(memory_space=pltpu.VMEM),
        ],
        out_specs=pl.BlockSpec(memory_space=pltpu.VMEM),
        out_shape=jax.ShapeDtypeStruct((G, FC), jnp.float32),
        scratch_shapes=[
            pltpu.VMEM((N, DIN), jnp.float32),
            pltpu.SemaphoreType.DMA,
        ],
    )(x, batch2d, W_atom, b_atom.reshape(1, FC))


# speedup vs baseline: 2.0228x; 1.0574x over previous
import jax
import jax.numpy as jnp
from jax.experimental import pallas as pl
from jax.experimental.pallas import tpu as pltpu

N = 10000
DIN = 128
FC = 256
G = 64


def _pool_kernel(x_hbm, batch_ref, w_ref, b_ref, out_ref, buf, sem):
    pltpu.make_async_copy(x_hbm, buf, sem).start()
    bat = batch_ref[...]
    lane = jax.lax.broadcasted_iota(jnp.int32, (1, G), 1)
    ends = jnp.zeros((1, G), jnp.int32)
    starts = jnp.zeros((1, G), jnp.int32)
    prev = jnp.zeros((), jnp.int32)
    for g in range(G):
        sel = (lane == g).astype(jnp.int32)
        e_g = jnp.sum((bat <= g).astype(jnp.int32))
        ends = ends + e_g * sel
        starts = starts + prev * sel
        prev = e_g
    riota = jax.lax.broadcasted_iota(jnp.int32, (N, G), 0)
    seg = ((riota >= starts) & (riota < ends)).astype(jnp.float32)
    cnt = (ends - starts).astype(jnp.float32)
    pltpu.make_async_copy(x_hbm, buf, sem).wait()
    sums = jax.lax.dot_general(seg, buf[...], (((0,), (0,)), ((), ())),
                               preferred_element_type=jnp.float32)
    pooled = sums / jnp.maximum(cnt, 1.0).T
    out_ref[...] = jnp.dot(pooled, w_ref[...],
                           preferred_element_type=jnp.float32) + b_ref[...]


def kernel(x, edge_index, edge_attr, batch, W_atom, b_atom, W_edge, b_edge):
    del edge_index, edge_attr, W_edge, b_edge
    batch2d = batch.reshape(80, 125)
    return pl.pallas_call(
        _pool_kernel,
        in_specs=[
            pl.BlockSpec(memory_space=pltpu.HBM),
            pl.BlockSpec(memory_space=pltpu.VMEM),
            pl.BlockSpec(memory_space=pltpu.VMEM),
            pl.BlockSpec(memory_space=pltpu.VMEM),
        ],
        out_specs=pl.BlockSpec(memory_space=pltpu.VMEM),
        out_shape=jax.ShapeDtypeStruct((G, FC), jnp.float32),
        scratch_shapes=[
            pltpu.VMEM((N, DIN), jnp.float32),
            pltpu.SemaphoreType.DMA,
        ],
    )(x, batch2d, W_atom, b_atom.reshape(1, FC))


# speedup vs baseline: 2.0420x; 1.0095x over previous
import jax
import jax.numpy as jnp
from jax.experimental import pallas as pl
from jax.experimental.pallas import tpu as pltpu

N = 10000
DIN = 128
FC = 256
G = 64


def _pool_kernel(x_hbm, batch_ref, w_ref, b_ref, out_ref, buf, sem):
    pltpu.make_async_copy(x_hbm, buf, sem).start()
    bat = batch_ref[...]
    lane = jax.lax.broadcasted_iota(jnp.int32, (1, G), 1)
    ends = jnp.zeros((1, G), jnp.int32)
    for g in range(G):
        sel = (lane == g).astype(jnp.int32)
        ends = ends + jnp.sum((bat <= g).astype(jnp.int32)) * sel
    starts = jnp.concatenate(
        [jnp.zeros((1, 1), jnp.int32), ends[:, :G - 1]], axis=1)
    riota = jax.lax.broadcasted_iota(jnp.int32, (N, G), 0)
    seg = ((riota >= starts) & (riota < ends)).astype(jnp.float32)
    cnt = (ends - starts).astype(jnp.float32)
    pltpu.make_async_copy(x_hbm, buf, sem).wait()
    sums = jax.lax.dot_general(seg, buf[...], (((0,), (0,)), ((), ())),
                               preferred_element_type=jnp.float32)
    pooled = sums / jnp.maximum(cnt, 1.0).T
    out_ref[...] = jnp.dot(pooled, w_ref[...],
                           preferred_element_type=jnp.float32) + b_ref[...]


def kernel(x, edge_index, edge_attr, batch, W_atom, b_atom, W_edge, b_edge):
    del edge_index, edge_attr, W_edge, b_edge
    batch2d = batch.reshape(80, 125)
    return pl.pallas_call(
        _pool_kernel,
        in_specs=[
            pl.BlockSpec(memory_space=pltpu.HBM),
            pl.BlockSpec(memory_space=pltpu.VMEM),
            pl.BlockSpec(memory_space=pltpu.VMEM),
            pl.BlockSpec(memory_space=pltpu.VMEM),
        ],
        out_specs=pl.BlockSpec(memory_space=pltpu.VMEM),
        out_shape=jax.ShapeDtypeStruct((G, FC), jnp.float32),
        scratch_shapes=[
            pltpu.VMEM((N, DIN), jnp.float32),
            pltpu.SemaphoreType.DMA,
        ],
    )(x, batch2d, W_atom, b_atom.reshape(1, FC))
